# baseline (device time: 28815 ns/iter reference)
import jax
import jax.numpy as jnp
from jax import lax
from jax.experimental import pallas as pl
from jax.experimental.pallas import tpu as pltpu

B, S, H_SHARD, D = 4, 512, 8, 64
K = H_SHARD * D
N = 1024
S_HALF = S // 2
R = 128
RZ = 64
NB = B // 2
XC = NB * 2
ZC = NB * 4


def kernel(O, Wo):
    O2 = O.reshape(B * S, H_SHARD, D)
    Od = lax.dynamic_slice_in_dim(
        O2, lax.axis_index("z") * (NB * S), NB * S, axis=0
    )

    def body(o_ref, wo_ref, out_ref, send_x, recv_x, send_z, recv_z,
             sx_sems, rx_sems, sz_sems, rz_sems):
        my_x = lax.axis_index("x")
        my_y = lax.axis_index("y")
        my_z = lax.axis_index("z")
        px = 1 - my_x
        pz = 1 - my_z

        barrier_sem = pltpu.get_barrier_semaphore()
        pl.semaphore_signal(
            barrier_sem, inc=1,
            device_id=(px, my_y, my_z), device_id_type=pl.DeviceIdType.MESH,
        )
        pl.semaphore_signal(
            barrier_sem, inc=1,
            device_id=(my_x, my_y, pz), device_id_type=pl.DeviceIdType.MESH,
        )
        pl.semaphore_wait(barrier_sem, 2)

        w = wo_ref[...].astype(jnp.bfloat16)
        partner_s = px * S_HALF
        my_s = my_x * S_HALF
        base_b = NB * my_z
        other_b = NB * pz

        x_rdmas = []
        for c in range(XC):
            i, sub = c // 2, c % 2
            row0 = i * S + partner_s + sub * R
            ob = (
                o_ref[pl.ds(row0, R), :, :]
                .reshape(R, K)
                .astype(jnp.bfloat16)
            )
            send_x[c, :, :] = jnp.dot(
                ob, w, preferred_element_type=jnp.float32
            ).astype(jnp.bfloat16)
            rdma = pltpu.make_async_remote_copy(
                src_ref=send_x.at[c],
                dst_ref=recv_x.at[c],
                send_sem=sx_sems.at[c],
                recv_sem=rx_sems.at[c],
                device_id=(px, my_y, my_z),
                device_id_type=pl.DeviceIdType.MESH,
            )
            rdma.start()
            x_rdmas.append(rdma)

        z_rdmas = []
        for i in range(NB):
            b = base_b + i
            ob = (
                o_ref[pl.ds(i * S + my_s, S_HALF), :, :]
                .reshape(S_HALF, K)
                .astype(jnp.bfloat16)
            )
            mine = jnp.dot(ob, w, preferred_element_type=jnp.float32)
            for sub in range(2):
                c = 2 * i + sub
                x_rdmas[c].wait_recv()
                fin = (
                    mine[sub * R:(sub + 1) * R, :]
                    + recv_x[c, :, :].astype(jnp.float32)
                ).astype(jnp.bfloat16)
                out_ref[pl.ds(b * S_HALF + sub * R, R), :] = fin
                for h in range(2):
                    zc = 2 * c + h
                    send_z[zc, :, :] = fin[h * RZ:(h + 1) * RZ, :]
                    rdma = pltpu.make_async_remote_copy(
                        src_ref=send_z.at[zc],
                        dst_ref=recv_z.at[zc],
                        send_sem=sz_sems.at[zc],
                        recv_sem=rz_sems.at[zc],
                        device_id=(my_x, my_y, pz),
                        device_id_type=pl.DeviceIdType.MESH,
                    )
                    rdma.start()
                    z_rdmas.append(rdma)

        for zc in range(ZC):
            i = zc // 4
            b = other_b + i
            off = (zc % 4) * RZ
            z_rdmas[zc].wait_recv()
            out_ref[pl.ds(b * S_HALF + off, RZ), :] = recv_z[zc, :, :]

        for c in range(XC):
            x_rdmas[c].wait_send()
        for zc in range(ZC):
            z_rdmas[zc].wait_send()

    out2 = pl.pallas_call(
        body,
        out_shape=jax.ShapeDtypeStruct((B * S_HALF, N), jnp.bfloat16),
        in_specs=[
            pl.BlockSpec(memory_space=pltpu.MemorySpace.VMEM),
            pl.BlockSpec(memory_space=pltpu.MemorySpace.VMEM),
        ],
        out_specs=pl.BlockSpec(memory_space=pltpu.MemorySpace.VMEM),
        scratch_shapes=[
            pltpu.VMEM((XC, R, N), jnp.bfloat16),
            pltpu.VMEM((XC, R, N), jnp.bfloat16),
            pltpu.VMEM((ZC, RZ, N), jnp.bfloat16),
            pltpu.VMEM((ZC, RZ, N), jnp.bfloat16),
            pltpu.SemaphoreType.DMA((XC,)),
            pltpu.SemaphoreType.DMA((XC,)),
            pltpu.SemaphoreType.DMA((ZC,)),
            pltpu.SemaphoreType.DMA((ZC,)),
        ],
        compiler_params=pltpu.CompilerParams(collective_id=0),
    )(Od, Wo)
    return out2.reshape(B, S_HALF, N)


# device time: 27652 ns/iter; 1.0421x vs baseline; 1.0421x over previous
import jax
import jax.numpy as jnp
from jax import lax
from jax.experimental import pallas as pl
from jax.experimental.pallas import tpu as pltpu

B, S, H_SHARD, D = 4, 512, 8, 64
K = H_SHARD * D
N = 1024
S_HALF = S // 2
R = 128
RZ = 64
NB = B // 2
XC = NB * 2
ZC = NB * 4


def kernel(O, Wo):
    O2 = O.reshape(B * S, H_SHARD, D)

    def body(o_ref, wo_ref, out_ref, send_x, recv_x, send_z, recv_z,
             sx_sems, rx_sems, sz_sems, rz_sems):
        my_x = lax.axis_index("x")
        my_y = lax.axis_index("y")
        my_z = lax.axis_index("z")
        px = 1 - my_x
        pz = 1 - my_z

        barrier_sem = pltpu.get_barrier_semaphore()
        pl.semaphore_signal(
            barrier_sem, inc=1,
            device_id=(px, my_y, my_z), device_id_type=pl.DeviceIdType.MESH,
        )
        pl.semaphore_signal(
            barrier_sem, inc=1,
            device_id=(my_x, my_y, pz), device_id_type=pl.DeviceIdType.MESH,
        )
        pl.semaphore_wait(barrier_sem, 2)

        w = wo_ref[...].astype(jnp.bfloat16)
        partner_s = px * S_HALF
        my_s = my_x * S_HALF
        base_b = NB * my_z
        other_b = NB * pz

        x_rdmas = []
        for c in range(XC):
            i, sub = c // 2, c % 2
            row0 = (base_b + i) * S + partner_s + sub * R
            ob = (
                o_ref[pl.ds(row0, R), :, :]
                .reshape(R, K)
                .astype(jnp.bfloat16)
            )
            send_x[c, :, :] = jnp.dot(
                ob, w, preferred_element_type=jnp.float32
            ).astype(jnp.bfloat16)
            rdma = pltpu.make_async_remote_copy(
                src_ref=send_x.at[c],
                dst_ref=recv_x.at[c],
                send_sem=sx_sems.at[c],
                recv_sem=rx_sems.at[c],
                device_id=(px, my_y, my_z),
                device_id_type=pl.DeviceIdType.MESH,
            )
            rdma.start()
            x_rdmas.append(rdma)

        z_rdmas = []
        for i in range(NB):
            b = base_b + i
            ob = (
                o_ref[pl.ds(b * S + my_s, S_HALF), :, :]
                .reshape(S_HALF, K)
                .astype(jnp.bfloat16)
            )
            mine = jnp.dot(ob, w, preferred_element_type=jnp.float32)
            for sub in range(2):
                c = 2 * i + sub
                x_rdmas[c].wait_recv()
                fin = (
                    mine[sub * R:(sub + 1) * R, :]
                    + recv_x[c, :, :].astype(jnp.float32)
                ).astype(jnp.bfloat16)
                out_ref[pl.ds(b * S_HALF + sub * R, R), :] = fin
                for h in range(2):
                    zc = 2 * c + h
                    send_z[zc, :, :] = fin[h * RZ:(h + 1) * RZ, :]
                    rdma = pltpu.make_async_remote_copy(
                        src_ref=send_z.at[zc],
                        dst_ref=recv_z.at[zc],
                        send_sem=sz_sems.at[zc],
                        recv_sem=rz_sems.at[zc],
                        device_id=(my_x, my_y, pz),
                        device_id_type=pl.DeviceIdType.MESH,
                    )
                    rdma.start()
                    z_rdmas.append(rdma)

        for zc in range(ZC):
            i = zc // 4
            b = other_b + i
            off = (zc % 4) * RZ
            z_rdmas[zc].wait_recv()
            out_ref[pl.ds(b * S_HALF + off, RZ), :] = recv_z[zc, :, :]

        for c in range(XC):
            x_rdmas[c].wait_send()
        for zc in range(ZC):
            z_rdmas[zc].wait_send()

    out2 = pl.pallas_call(
        body,
        out_shape=jax.ShapeDtypeStruct((B * S_HALF, N), jnp.bfloat16),
        in_specs=[
            pl.BlockSpec(memory_space=pltpu.MemorySpace.VMEM),
            pl.BlockSpec(memory_space=pltpu.MemorySpace.VMEM),
        ],
        out_specs=pl.BlockSpec(memory_space=pltpu.MemorySpace.VMEM),
        scratch_shapes=[
            pltpu.VMEM((XC, R, N), jnp.bfloat16),
            pltpu.VMEM((XC, R, N), jnp.bfloat16),
            pltpu.VMEM((ZC, RZ, N), jnp.bfloat16),
            pltpu.VMEM((ZC, RZ, N), jnp.bfloat16),
            pltpu.SemaphoreType.DMA((XC,)),
            pltpu.SemaphoreType.DMA((XC,)),
            pltpu.SemaphoreType.DMA((ZC,)),
            pltpu.SemaphoreType.DMA((ZC,)),
        ],
        compiler_params=pltpu.CompilerParams(collective_id=0),
    )(O2, Wo)
    return out2.reshape(B, S_HALF, N)


# device time: 26930 ns/iter; 1.0700x vs baseline; 1.0268x over previous
import jax
import jax.numpy as jnp
from jax import lax
from jax.experimental import pallas as pl
from jax.experimental.pallas import tpu as pltpu

B, S, H_SHARD, D = 4, 512, 8, 64
K = H_SHARD * D
N = 1024
S_HALF = S // 2
R = 64
RZ = 64
NB = B // 2
XC = NB * 4
ZC = NB * 4


def kernel(O, Wo):
    O2 = O.reshape(B * S, H_SHARD, D)

    def body(o_ref, wo_ref, out_ref, send_x, recv_x, send_z, recv_z,
             sx_sems, rx_sems, sz_sems, rz_sems):
        my_x = lax.axis_index("x")
        my_y = lax.axis_index("y")
        my_z = lax.axis_index("z")
        px = 1 - my_x
        pz = 1 - my_z

        barrier_sem = pltpu.get_barrier_semaphore()
        pl.semaphore_signal(
            barrier_sem, inc=1,
            device_id=(px, my_y, my_z), device_id_type=pl.DeviceIdType.MESH,
        )
        pl.semaphore_signal(
            barrier_sem, inc=1,
            device_id=(my_x, my_y, pz), device_id_type=pl.DeviceIdType.MESH,
        )
        pl.semaphore_wait(barrier_sem, 2)

        w = wo_ref[...].astype(jnp.bfloat16)
        partner_s = px * S_HALF
        my_s = my_x * S_HALF
        base_b = NB * my_z
        other_b = NB * pz

        x_rdmas = []
        for c in range(XC):
            i, sub = c // 4, c % 4
            row0 = (base_b + i) * S + partner_s + sub * R
            ob = (
                o_ref[pl.ds(row0, R), :, :]
                .reshape(R, K)
                .astype(jnp.bfloat16)
            )
            send_x[c, :, :] = jnp.dot(
                ob, w, preferred_element_type=jnp.float32
            ).astype(jnp.bfloat16)
            rdma = pltpu.make_async_remote_copy(
                src_ref=send_x.at[c],
                dst_ref=recv_x.at[c],
                send_sem=sx_sems.at[c],
                recv_sem=rx_sems.at[c],
                device_id=(px, my_y, my_z),
                device_id_type=pl.DeviceIdType.MESH,
            )
            rdma.start()
            x_rdmas.append(rdma)

        z_rdmas = []
        for i in range(NB):
            b = base_b + i
            ob = (
                o_ref[pl.ds(b * S + my_s, S_HALF), :, :]
                .reshape(S_HALF, K)
                .astype(jnp.bfloat16)
            )
            mine = jnp.dot(ob, w, preferred_element_type=jnp.float32)
            for sub in range(4):
                c = 4 * i + sub
                x_rdmas[c].wait_recv()
                fin = (
                    mine[sub * R:(sub + 1) * R, :]
                    + recv_x[c, :, :].astype(jnp.float32)
                ).astype(jnp.bfloat16)
                out_ref[pl.ds(b * S_HALF + sub * R, R), :] = fin
                send_z[c, :, :] = fin
                rdma = pltpu.make_async_remote_copy(
                    src_ref=send_z.at[c],
                    dst_ref=recv_z.at[c],
                    send_sem=sz_sems.at[c],
                    recv_sem=rz_sems.at[c],
                    device_id=(my_x, my_y, pz),
                    device_id_type=pl.DeviceIdType.MESH,
                )
                rdma.start()
                z_rdmas.append(rdma)

        for zc in range(ZC):
            i = zc // 4
            b = other_b + i
            off = (zc % 4) * RZ
            z_rdmas[zc].wait_recv()
            out_ref[pl.ds(b * S_HALF + off, RZ), :] = recv_z[zc, :, :]

        for c in range(XC):
            x_rdmas[c].wait_send()
        for zc in range(ZC):
            z_rdmas[zc].wait_send()

    out2 = pl.pallas_call(
        body,
        out_shape=jax.ShapeDtypeStruct((B * S_HALF, N), jnp.bfloat16),
        in_specs=[
            pl.BlockSpec(memory_space=pltpu.MemorySpace.VMEM),
            pl.BlockSpec(memory_space=pltpu.MemorySpace.VMEM),
        ],
        out_specs=pl.BlockSpec(memory_space=pltpu.MemorySpace.VMEM),
        scratch_shapes=[
            pltpu.VMEM((XC, R, N), jnp.bfloat16),
            pltpu.VMEM((XC, R, N), jnp.bfloat16),
            pltpu.VMEM((ZC, RZ, N), jnp.bfloat16),
            pltpu.VMEM((ZC, RZ, N), jnp.bfloat16),
            pltpu.SemaphoreType.DMA((XC,)),
            pltpu.SemaphoreType.DMA((XC,)),
            pltpu.SemaphoreType.DMA((ZC,)),
            pltpu.SemaphoreType.DMA((ZC,)),
        ],
        compiler_params=pltpu.CompilerParams(collective_id=0),
    )(O2, Wo)
    return out2.reshape(B, S_HALF, N)
